# baseline (speedup 1.0000x reference)
import functools

import jax
import jax.numpy as jnp
import numpy as np
from jax import lax
from jax.experimental import pallas as pl
from jax.experimental.pallas import tpu as pltpu
from jax.experimental.pallas import tpu_sc as plsc


def kernel(a, b, add_in_width0, add_in_width1):
    T1, B, W1, Hh1, Hw1 = a.shape
    W2, H2 = b.shape
    H1 = Hh1 * Hw1
    buffer_size = 32 * 1024
    max_h1 = 256
    max_w2 = 256

    cond_elif = (H1 <= max_h1) or (W2 <= max_w2)
    ceil_w2 = float(np.ceil(W2 / 256))
    load_first_aw0_coef = float(H1 * W1 * T1) / 32.0
    load_aw1_coef = float(H2 * W2) / 32.0
    load_elif_aw0_coef = float(H1 * W1 * T1) * ceil_w2 / 32.0
    compute_lat = float((1 + H1 + 4) * np.ceil(W1 / 16) * ceil_w2) * T1
    lif_lat = float(H1) * ceil_w2 * T1
    store_lat = float(H1 * W2) / 32.0 * T1

    mesh = plsc.ScalarSubcoreMesh(axis_name="c", num_cores=2)

    @functools.partial(
        pl.kernel,
        mesh=mesh,
        out_type=jax.ShapeDtypeStruct((8,), jnp.float32),
        scratch_types=[pltpu.SMEM((8,), jnp.int32),
                       pltpu.SMEM((8,), jnp.float32)],
    )
    def latency_sc(aw_hbm, out_hbm, aw_smem, res_smem):
        cid = lax.axis_index("c")

        @pl.when(cid == 0)
        def _():
            pltpu.sync_copy(aw_hbm, aw_smem)
            aw0 = aw_smem[0].astype(jnp.float32)
            aw1 = aw_smem[1].astype(jnp.float32)
            denom = 32.0 * aw0 + 2.0 * aw1
            fbuf = float(buffer_size)
            cond_first = jnp.logical_or(
                jnp.logical_and(H1 <= max_h1, W1 * denom <= fbuf),
                jnp.logical_and(H2 * denom <= fbuf, W2 <= max_w2),
            )
            active = jnp.logical_or(cond_first, cond_elif)
            load_first = load_first_aw0_coef * aw0 + load_aw1_coef * aw1
            load_elif = load_elif_aw0_coef * aw0 + load_aw1_coef * aw1
            load_latency = jnp.where(
                cond_first, load_first,
                jnp.where(cond_elif, load_elif, 0.0),
            )
            compute_latency = jnp.where(active, compute_lat, 0.0)
            lif_latency = jnp.where(active, lif_lat, 0.0)
            store_latency = jnp.where(active, store_lat, 0.0)
            latency_a = (load_latency + compute_latency
                         + lif_latency + store_latency)
            res_smem[0] = latency_a * B
            res_smem[1] = load_latency * B
            res_smem[2] = compute_latency * B
            res_smem[3] = lif_latency * B
            res_smem[4] = store_latency * B
            res_smem[5] = 0.0
            res_smem[6] = 0.0
            res_smem[7] = 0.0
            pltpu.sync_copy(res_smem, out_hbm)

    aw_vec = jnp.asarray(
        [add_in_width0, add_in_width1, 0, 0, 0, 0, 0, 0], jnp.int32)
    out = latency_sc(aw_vec)
    return (out[0], out[1], out[2], out[3], out[4])


# speedup vs baseline: 7.9718x; 7.9718x over previous
import jax
import jax.numpy as jnp
import numpy as np
from jax.experimental import pallas as pl
from jax.experimental.pallas import tpu as pltpu


def kernel(a, b, add_in_width0, add_in_width1):
    T1, B, W1, Hh1, Hw1 = a.shape
    W2, H2 = b.shape
    H1 = Hh1 * Hw1
    buffer_size = 32 * 1024
    max_h1 = 256
    max_w2 = 256

    cond_elif = (H1 <= max_h1) or (W2 <= max_w2)
    ceil_w2 = float(np.ceil(W2 / 256))
    load_first_aw0_coef = float(H1 * W1 * T1) / 32.0
    load_aw1_coef = float(H2 * W2) / 32.0
    load_elif_aw0_coef = float(H1 * W1 * T1) * ceil_w2 / 32.0
    compute_lat = float((1 + H1 + 4) * np.ceil(W1 / 16) * ceil_w2) * T1
    lif_lat = float(H1) * ceil_w2 * T1
    store_lat = float(H1 * W2) / 32.0 * T1

    def body(aw0_ref, aw1_ref, *out_refs):
        aw0 = aw0_ref[...].astype(jnp.float32)
        aw1 = aw1_ref[...].astype(jnp.float32)
        max_w1_h2 = buffer_size / (256.0 * aw0 / 8.0 + 16.0 * aw1 / 8.0)
        cond_first = jnp.logical_or(
            jnp.logical_and(H1 <= max_h1, W1 <= max_w1_h2),
            jnp.logical_and(H2 <= max_w1_h2, W2 <= max_w2),
        )
        active = jnp.logical_or(cond_first, cond_elif)
        load_first = load_first_aw0_coef * aw0 + load_aw1_coef * aw1
        load_elif = load_elif_aw0_coef * aw0 + load_aw1_coef * aw1
        load_latency = jnp.where(
            cond_first, load_first,
            jnp.where(cond_elif, load_elif, 0.0),
        )
        compute_latency = jnp.where(active, compute_lat, 0.0)
        lif_latency = jnp.where(active, lif_lat, 0.0)
        store_latency = jnp.where(active, store_lat, 0.0)
        latency_a = (load_latency + compute_latency
                     + lif_latency + store_latency)
        out_refs[0][...] = latency_a * B
        out_refs[1][...] = load_latency * B
        out_refs[2][...] = compute_latency * B
        out_refs[3][...] = lif_latency * B
        out_refs[4][...] = store_latency * B

    return pl.pallas_call(
        body,
        out_shape=tuple(
            jax.ShapeDtypeStruct((), jnp.float32) for _ in range(5)),
        in_specs=[pl.BlockSpec(memory_space=pltpu.SMEM)] * 2,
        out_specs=tuple(pl.BlockSpec(memory_space=pltpu.SMEM)
                        for _ in range(5)),
    )(add_in_width0, add_in_width1)
